# baseline (speedup 1.0000x reference)
import jax
import jax.numpy as jnp
from jax import lax
from jax.experimental import pallas as pl
from jax.experimental.pallas import tpu as pltpu
from jax.experimental.pallas import tpu_sc as plsc

_N = 10000
_D = 128
_O = 2
_NC = 2
_NS = 16
_K = 128
_NPAD = 10240
_RPT = _NPAD // _NS


def _sc_mesh():
    return plsc.VectorSubcoreMesh(core_axis_name="c", subcore_axis_name="s",
                                  num_cores=_NC, num_subcores=_NS)


def _make_deg_kernel(e_pad):
    cpt = e_pad // (_NC * _NS * _K)

    def body(dst_hbm, zeros_hbm, ones_hbm, deg_out, d0, ones_v, deg_sh):
        c = lax.axis_index("c")
        s = lax.axis_index("s")
        r0 = s * _RPT
        base = (c * _NS + s) * cpt * _K
        pltpu.sync_copy(zeros_hbm.at[pl.ds(r0, _RPT)], deg_sh.at[pl.ds(r0, _RPT)])
        pltpu.sync_copy(ones_hbm, ones_v)
        plsc.subcore_barrier()

        def chunk(i, carry):
            off = base + i * _K
            pltpu.sync_copy(dst_hbm.at[pl.ds(off, _K)], d0)
            pltpu.sync_copy(ones_v, deg_sh.at[d0], add=True)
            return carry

        lax.fori_loop(0, cpt, chunk, 0)
        plsc.subcore_barrier()
        pltpu.sync_copy(deg_sh.at[pl.ds(r0, _RPT)],
                        deg_out.at[c, pl.ds(r0, _RPT)])

    return pl.kernel(
        body,
        out_type=jax.ShapeDtypeStruct((_NC, _NPAD), jnp.float32),
        mesh=_sc_mesh(),
        scratch_types=[
            pltpu.VMEM((_K,), jnp.int32),
            pltpu.VMEM((_K,), jnp.float32),
            pltpu.VMEM_SHARED((_NPAD,), jnp.float32),
        ],
    )


def _mm_body(x_ref, w1_ref, h_ref):
    h_ref[0:_N, :] = jnp.dot(x_ref[...], w1_ref[...],
                             preferred_element_type=jnp.float32)
    h_ref[_N:_NPAD, :] = jnp.zeros((_NPAD - _N, _D), jnp.float32)


def _mm_call(x, w1):
    return pl.pallas_call(
        _mm_body,
        out_shape=jax.ShapeDtypeStruct((_NPAD, _D), jnp.float32),
    )(x, w1)


def _scale_body(h_ref, degt_ref, dinv_ref, hs_ref):
    deg = degt_ref[:, 0:1] + degt_ref[:, 1:2] + 1.0
    dinv = lax.rsqrt(deg)
    dinv_ref[...] = dinv
    hs_ref[...] = h_ref[...] * dinv


def _scale_call(h, degt):
    return pl.pallas_call(
        _scale_body,
        out_shape=(
            jax.ShapeDtypeStruct((_NPAD, 1), jnp.float32),
            jax.ShapeDtypeStruct((_NPAD, _D), jnp.float32),
        ),
    )(h, degt)


def _make_agg_kernel(e_pad):
    cpt = e_pad // (_NC * _NS * _K)

    def body(hs_hbm, src_hbm, dst_hbm, zeros2_hbm, agg_out,
             s0, d0, rows0, agg_sh):
        c = lax.axis_index("c")
        s = lax.axis_index("s")
        base = (c * _NS + s) * cpt * _K
        pltpu.sync_copy(zeros2_hbm, rows0)
        for j in range(_RPT // _K):
            pltpu.sync_copy(rows0, agg_sh.at[pl.ds(s * _RPT + j * _K, _K)])
        plsc.subcore_barrier()

        def chunk(i, carry):
            off = base + i * _K
            pltpu.sync_copy(src_hbm.at[pl.ds(off, _K)], s0)
            pltpu.sync_copy(dst_hbm.at[pl.ds(off, _K)], d0)
            pltpu.sync_copy(hs_hbm.at[s0], rows0)
            pltpu.sync_copy(rows0, agg_sh.at[d0], add=True)
            return carry

        lax.fori_loop(0, cpt, chunk, 0)
        plsc.subcore_barrier()
        r0 = s * _RPT
        pltpu.sync_copy(agg_sh.at[pl.ds(r0, _RPT)],
                        agg_out.at[c, pl.ds(r0, _RPT)])

    return pl.kernel(
        body,
        out_type=jax.ShapeDtypeStruct((_NC, _NPAD, _D), jnp.float32),
        mesh=_sc_mesh(),
        scratch_types=[
            pltpu.VMEM((_K,), jnp.int32),
            pltpu.VMEM((_K,), jnp.int32),
            pltpu.VMEM((_K, _D), jnp.float32),
            pltpu.VMEM_SHARED((_NPAD, _D), jnp.float32),
        ],
    )


def _dense2_body(aggp_ref, hs_ref, dinv_ref, b1_ref, gamma_ref, beta_ref,
                 w2_ref, b2_ref, prob_ref, rsu_ref):
    a = aggp_ref[0, 0:_N, :] + aggp_ref[1, 0:_N, :] + hs_ref[0:_N, :]
    y = a * dinv_ref[0:_N, :] + b1_ref[...]
    mean = jnp.mean(y, axis=0, keepdims=True)
    d = y - mean
    var = jnp.mean(d * d, axis=0, keepdims=True)
    bn = d * lax.rsqrt(var + 1e-5) * gamma_ref[...] + beta_ref[...]
    rsu_ref[...] = bn[0:1, :]
    z = jnp.dot(bn, w2_ref[...], preferred_element_type=jnp.float32)
    z = jnp.maximum(z + b2_ref[...], 0.0)
    m = jnp.max(z, axis=1, keepdims=True)
    e = jnp.exp(z - m)
    prob_ref[...] = e / jnp.sum(e, axis=1, keepdims=True)


def _dense2_call(aggp, hs, dinv, b1, gamma, beta, w2, b2):
    return pl.pallas_call(
        _dense2_body,
        out_shape=(
            jax.ShapeDtypeStruct((_N, _O), jnp.float32),
            jax.ShapeDtypeStruct((1, _D), jnp.float32),
        ),
    )(aggp, hs, dinv, b1, gamma, beta, w2, b2)


def kernel(node_feature, edge_index, W1, b1, gamma, beta, W2, b2):
    e = edge_index.shape[1]
    chunk = _NC * _NS * _K
    e_pad = ((e + chunk - 1) // chunk) * chunk
    pad = jnp.full((e_pad - e,), _N, jnp.int32)
    src = jnp.concatenate([edge_index[0], pad])
    dst = jnp.concatenate([edge_index[1], pad])

    zeros1 = jnp.zeros((_NPAD,), jnp.float32)
    ones_k = jnp.ones((_K,), jnp.float32)
    zeros2 = jnp.zeros((_K, _D), jnp.float32)

    degp = _make_deg_kernel(e_pad)(dst, zeros1, ones_k)
    h = _mm_call(node_feature, W1)
    degt = degp.T
    dinv, hs = _scale_call(h, degt)
    aggp = _make_agg_kernel(e_pad)(hs, src, dst, zeros2)
    prob, rsu = _dense2_call(aggp, hs, dinv, b1, gamma, beta, W2, b2)
    return (prob, rsu)


# speedup vs baseline: 1.1120x; 1.1120x over previous
import jax
import jax.numpy as jnp
from jax import lax
from jax.experimental import pallas as pl
from jax.experimental.pallas import tpu as pltpu
from jax.experimental.pallas import tpu_sc as plsc

_N = 10000
_D = 128
_O = 2
_NC = 2
_NS = 16
_K = 128
_NPAD = 10240
_RPT = _NPAD // _NS


def _sc_mesh():
    return plsc.VectorSubcoreMesh(core_axis_name="c", subcore_axis_name="s",
                                  num_cores=_NC, num_subcores=_NS)


def _make_deg_kernel(e_pad):
    cpt = e_pad // (_NC * _NS * _K)

    def body(dst_hbm, zeros_hbm, ones_hbm, deg_out, d0, ones_v, deg_sh):
        c = lax.axis_index("c")
        s = lax.axis_index("s")
        r0 = s * _RPT
        base = (c * _NS + s) * cpt * _K
        pltpu.sync_copy(zeros_hbm.at[pl.ds(r0, _RPT)], deg_sh.at[pl.ds(r0, _RPT)])
        pltpu.sync_copy(ones_hbm, ones_v)
        plsc.subcore_barrier()

        def chunk(i, carry):
            off = base + i * _K
            pltpu.sync_copy(dst_hbm.at[pl.ds(off, _K)], d0)
            pltpu.sync_copy(ones_v, deg_sh.at[d0], add=True)
            return carry

        lax.fori_loop(0, cpt, chunk, 0)
        plsc.subcore_barrier()
        pltpu.sync_copy(deg_sh.at[pl.ds(r0, _RPT)],
                        deg_out.at[c, pl.ds(r0, _RPT)])

    return pl.kernel(
        body,
        out_type=jax.ShapeDtypeStruct((_NC, _NPAD), jnp.float32),
        mesh=_sc_mesh(),
        scratch_types=[
            pltpu.VMEM((_K,), jnp.int32),
            pltpu.VMEM((_K,), jnp.float32),
            pltpu.VMEM_SHARED((_NPAD,), jnp.float32),
        ],
    )


def _mm_body(x_ref, w1_ref, h_ref):
    h_ref[0:_N, :] = jnp.dot(x_ref[...], w1_ref[...],
                             preferred_element_type=jnp.float32)
    h_ref[_N:_NPAD, :] = jnp.zeros((_NPAD - _N, _D), jnp.float32)


def _mm_call(x, w1):
    return pl.pallas_call(
        _mm_body,
        out_shape=jax.ShapeDtypeStruct((_NPAD, _D), jnp.float32),
    )(x, w1)


def _scale_body(h_ref, degt_ref, dinv_ref, hs_ref):
    deg = degt_ref[:, 0:1] + degt_ref[:, 1:2] + 1.0
    dinv = lax.rsqrt(deg)
    dinv_ref[...] = dinv
    hs_ref[...] = h_ref[...] * dinv


def _scale_call(h, degt):
    return pl.pallas_call(
        _scale_body,
        out_shape=(
            jax.ShapeDtypeStruct((_NPAD, 1), jnp.float32),
            jax.ShapeDtypeStruct((_NPAD, _D), jnp.float32),
        ),
    )(h, degt)


_KA = 64


def _make_agg_kernel(e_pad):
    cpt = e_pad // (_NC * _NS * _KA)

    def body(hs_hbm, src_hbm, dst_hbm, zeros2_hbm, agg_out,
             s0, s1, d0, d1, rows0, rows1, agg_sh, g0, g1):
        c = lax.axis_index("c")
        s = lax.axis_index("s")
        base = (c * _NS + s) * cpt * _KA
        pltpu.sync_copy(zeros2_hbm, rows0)
        for j in range(_RPT // _KA):
            pltpu.sync_copy(rows0, agg_sh.at[pl.ds(s * _RPT + j * _KA, _KA)])
        plsc.subcore_barrier()

        pltpu.sync_copy(src_hbm.at[pl.ds(base, _KA)], s0)
        pltpu.sync_copy(dst_hbm.at[pl.ds(base, _KA)], d0)
        pltpu.async_copy(hs_hbm.at[s0], rows0, g0)

        def pair(j, carry):
            i1 = 2 * j + 1
            i2 = jnp.minimum(2 * j + 2, cpt - 1)
            pltpu.sync_copy(src_hbm.at[pl.ds(base + i1 * _KA, _KA)], s1)
            pltpu.sync_copy(dst_hbm.at[pl.ds(base + i1 * _KA, _KA)], d1)
            pltpu.make_async_copy(hs_hbm.at[s0], rows0, g0).wait()
            pltpu.async_copy(hs_hbm.at[s1], rows1, g1)
            pltpu.sync_copy(rows0, agg_sh.at[d0], add=True)
            pltpu.sync_copy(src_hbm.at[pl.ds(base + i2 * _KA, _KA)], s0)
            pltpu.sync_copy(dst_hbm.at[pl.ds(base + i2 * _KA, _KA)], d0)
            pltpu.make_async_copy(hs_hbm.at[s1], rows1, g1).wait()
            pltpu.async_copy(hs_hbm.at[s0], rows0, g0)
            pltpu.sync_copy(rows1, agg_sh.at[d1], add=True)
            return carry

        lax.fori_loop(0, cpt // 2, pair, 0)
        pltpu.make_async_copy(hs_hbm.at[s0], rows0, g0).wait()
        plsc.subcore_barrier()
        r0 = s * _RPT
        pltpu.sync_copy(agg_sh.at[pl.ds(r0, _RPT)],
                        agg_out.at[c, pl.ds(r0, _RPT)])

    return pl.kernel(
        body,
        out_type=jax.ShapeDtypeStruct((_NC, _NPAD, _D), jnp.float32),
        mesh=_sc_mesh(),
        scratch_types=[
            pltpu.VMEM((_KA,), jnp.int32),
            pltpu.VMEM((_KA,), jnp.int32),
            pltpu.VMEM((_KA,), jnp.int32),
            pltpu.VMEM((_KA,), jnp.int32),
            pltpu.VMEM((_KA, _D), jnp.float32),
            pltpu.VMEM((_KA, _D), jnp.float32),
            pltpu.VMEM_SHARED((_NPAD, _D), jnp.float32),
            pltpu.SemaphoreType.DMA,
            pltpu.SemaphoreType.DMA,
        ],
    )


def _dense2_body(aggp_ref, hs_ref, dinv_ref, b1_ref, gamma_ref, beta_ref,
                 w2_ref, b2_ref, prob_ref, rsu_ref):
    a = aggp_ref[0, 0:_N, :] + aggp_ref[1, 0:_N, :] + hs_ref[0:_N, :]
    y = a * dinv_ref[0:_N, :] + b1_ref[...]
    mean = jnp.mean(y, axis=0, keepdims=True)
    d = y - mean
    var = jnp.mean(d * d, axis=0, keepdims=True)
    bn = d * lax.rsqrt(var + 1e-5) * gamma_ref[...] + beta_ref[...]
    rsu_ref[...] = bn[0:1, :]
    z = jnp.dot(bn, w2_ref[...], preferred_element_type=jnp.float32)
    z = jnp.maximum(z + b2_ref[...], 0.0)
    m = jnp.max(z, axis=1, keepdims=True)
    e = jnp.exp(z - m)
    prob_ref[...] = e / jnp.sum(e, axis=1, keepdims=True)


def _dense2_call(aggp, hs, dinv, b1, gamma, beta, w2, b2):
    return pl.pallas_call(
        _dense2_body,
        out_shape=(
            jax.ShapeDtypeStruct((_N, _O), jnp.float32),
            jax.ShapeDtypeStruct((1, _D), jnp.float32),
        ),
    )(aggp, hs, dinv, b1, gamma, beta, w2, b2)


def kernel(node_feature, edge_index, W1, b1, gamma, beta, W2, b2):
    e = edge_index.shape[1]
    chunk = _NC * _NS * _K
    e_pad = ((e + chunk - 1) // chunk) * chunk
    pad = jnp.full((e_pad - e,), _N, jnp.int32)
    src = jnp.concatenate([edge_index[0], pad])
    dst = jnp.concatenate([edge_index[1], pad])

    zeros1 = jnp.zeros((_NPAD,), jnp.float32)
    ones_k = jnp.ones((_K,), jnp.float32)
    zeros2 = jnp.zeros((_KA, _D), jnp.float32)

    degp = _make_deg_kernel(e_pad)(dst, zeros1, ones_k)
    h = _mm_call(node_feature, W1)
    degt = degp.T
    dinv, hs = _scale_call(h, degt)
    aggp = _make_agg_kernel(e_pad)(hs, src, dst, zeros2)
    prob, rsu = _dense2_call(aggp, hs, dinv, b1, gamma, beta, W2, b2)
    return (prob, rsu)


# speedup vs baseline: 1.2924x; 1.1622x over previous
import jax
import jax.numpy as jnp
from jax import lax
from jax.experimental import pallas as pl
from jax.experimental.pallas import tpu as pltpu
from jax.experimental.pallas import tpu_sc as plsc

_N = 10000
_D = 128
_O = 2
_NC = 2
_NS = 16
_K = 128
_NPAD = 10240
_RPT = _NPAD // _NS


def _sc_mesh():
    return plsc.VectorSubcoreMesh(core_axis_name="c", subcore_axis_name="s",
                                  num_cores=_NC, num_subcores=_NS)


def _make_deg_kernel(e_pad):
    cpt = e_pad // (_NC * _NS * _K)

    def body(dst_hbm, zeros_hbm, ones_hbm, deg_out, d_all, ones_v, deg_sh):
        c = lax.axis_index("c")
        s = lax.axis_index("s")
        r0 = s * _RPT
        base = (c * _NS + s) * cpt * _K
        pltpu.sync_copy(zeros_hbm.at[pl.ds(r0, _RPT)], deg_sh.at[pl.ds(r0, _RPT)])
        pltpu.sync_copy(ones_hbm, ones_v)
        pltpu.sync_copy(dst_hbm.at[pl.ds(base, cpt * _K)], d_all)
        plsc.subcore_barrier()

        def chunk(i, carry):
            pltpu.sync_copy(ones_v, deg_sh.at[d_all.at[pl.ds(i * _K, _K)]],
                            add=True)
            return carry

        lax.fori_loop(0, cpt, chunk, 0)
        plsc.subcore_barrier()
        pltpu.sync_copy(deg_sh.at[pl.ds(r0, _RPT)],
                        deg_out.at[c, pl.ds(r0, _RPT)])

    return pl.kernel(
        body,
        out_type=jax.ShapeDtypeStruct((_NC, _NPAD), jnp.float32),
        mesh=_sc_mesh(),
        scratch_types=[
            pltpu.VMEM((cpt * _K,), jnp.int32),
            pltpu.VMEM((_K,), jnp.float32),
            pltpu.VMEM_SHARED((_NPAD,), jnp.float32),
        ],
    )


def _mm_body(x_ref, w1_ref, h_ref):
    h_ref[0:_N, :] = jnp.dot(x_ref[...], w1_ref[...],
                             preferred_element_type=jnp.float32)
    h_ref[_N:_NPAD, :] = jnp.zeros((_NPAD - _N, _D), jnp.float32)


def _mm_call(x, w1):
    return pl.pallas_call(
        _mm_body,
        out_shape=jax.ShapeDtypeStruct((_NPAD, _D), jnp.float32),
    )(x, w1)


def _scale_body(h_ref, degt_ref, dinv_ref, hs_ref):
    deg = degt_ref[:, 0:1] + degt_ref[:, 1:2] + 1.0
    dinv = lax.rsqrt(deg)
    dinv_ref[...] = dinv
    hs_ref[...] = h_ref[...] * dinv


def _scale_call(h, degt):
    return pl.pallas_call(
        _scale_body,
        out_shape=(
            jax.ShapeDtypeStruct((_NPAD, 1), jnp.float32),
            jax.ShapeDtypeStruct((_NPAD, _D), jnp.float32),
        ),
    )(h, degt)


_KA = 64


def _make_agg_kernel(e_pad):
    cpt = e_pad // (_NC * _NS * _KA)

    def body(hs_hbm, src_hbm, dst_hbm, zeros2_hbm, agg_out,
             s_all, d_all, rows0, rows1, agg_sh, g0, g1):
        c = lax.axis_index("c")
        s = lax.axis_index("s")
        base = (c * _NS + s) * cpt * _KA
        pltpu.sync_copy(zeros2_hbm, rows0)
        for j in range(_RPT // _KA):
            pltpu.sync_copy(rows0, agg_sh.at[pl.ds(s * _RPT + j * _KA, _KA)])
        plsc.subcore_barrier()

        pltpu.sync_copy(src_hbm.at[pl.ds(base, cpt * _KA)], s_all)
        pltpu.sync_copy(dst_hbm.at[pl.ds(base, cpt * _KA)], d_all)

        pltpu.async_copy(hs_hbm.at[s_all.at[pl.ds(0, _KA)]], rows0, g0)

        def pair(j, carry):
            o0 = 2 * j * _KA
            o1 = o0 + _KA
            o2 = jnp.minimum(o1 + _KA, (cpt - 1) * _KA)
            pltpu.make_async_copy(
                hs_hbm.at[s_all.at[pl.ds(o0, _KA)]], rows0, g0).wait()
            pltpu.async_copy(hs_hbm.at[s_all.at[pl.ds(o1, _KA)]], rows1, g1)
            pltpu.sync_copy(rows0, agg_sh.at[d_all.at[pl.ds(o0, _KA)]],
                            add=True)
            pltpu.make_async_copy(
                hs_hbm.at[s_all.at[pl.ds(o1, _KA)]], rows1, g1).wait()
            pltpu.async_copy(hs_hbm.at[s_all.at[pl.ds(o2, _KA)]], rows0, g0)
            pltpu.sync_copy(rows1, agg_sh.at[d_all.at[pl.ds(o1, _KA)]],
                            add=True)
            return carry

        lax.fori_loop(0, cpt // 2, pair, 0)
        pltpu.make_async_copy(
            hs_hbm.at[s_all.at[pl.ds(0, _KA)]], rows0, g0).wait()
        plsc.subcore_barrier()
        r0 = s * _RPT
        pltpu.sync_copy(agg_sh.at[pl.ds(r0, _RPT)],
                        agg_out.at[c, pl.ds(r0, _RPT)])

    def make(cpt):
        return pl.kernel(
            body,
            out_type=jax.ShapeDtypeStruct((_NC, _NPAD, _D), jnp.float32),
            mesh=_sc_mesh(),
            scratch_types=[
                pltpu.VMEM((cpt * _KA,), jnp.int32),
                pltpu.VMEM((cpt * _KA,), jnp.int32),
                pltpu.VMEM((_KA, _D), jnp.float32),
                pltpu.VMEM((_KA, _D), jnp.float32),
                pltpu.VMEM_SHARED((_NPAD, _D), jnp.float32),
                pltpu.SemaphoreType.DMA,
                pltpu.SemaphoreType.DMA,
            ],
        )

    return make(cpt)


def _dense2_body(aggp_ref, hs_ref, dinv_ref, b1_ref, gamma_ref, beta_ref,
                 w2_ref, b2_ref, prob_ref, rsu_ref):
    a = aggp_ref[0, 0:_N, :] + aggp_ref[1, 0:_N, :] + hs_ref[0:_N, :]
    y = a * dinv_ref[0:_N, :] + b1_ref[...]
    mean = jnp.mean(y, axis=0, keepdims=True)
    d = y - mean
    var = jnp.mean(d * d, axis=0, keepdims=True)
    bn = d * lax.rsqrt(var + 1e-5) * gamma_ref[...] + beta_ref[...]
    rsu_ref[...] = bn[0:1, :]
    z = jnp.dot(bn, w2_ref[...], preferred_element_type=jnp.float32)
    z = jnp.maximum(z + b2_ref[...], 0.0)
    m = jnp.max(z, axis=1, keepdims=True)
    e = jnp.exp(z - m)
    prob_ref[...] = e / jnp.sum(e, axis=1, keepdims=True)


def _dense2_call(aggp, hs, dinv, b1, gamma, beta, w2, b2):
    return pl.pallas_call(
        _dense2_body,
        out_shape=(
            jax.ShapeDtypeStruct((_N, _O), jnp.float32),
            jax.ShapeDtypeStruct((1, _D), jnp.float32),
        ),
    )(aggp, hs, dinv, b1, gamma, beta, w2, b2)


def kernel(node_feature, edge_index, W1, b1, gamma, beta, W2, b2):
    e = edge_index.shape[1]
    chunk = _NC * _NS * _K
    e_pad = ((e + chunk - 1) // chunk) * chunk
    pad = jnp.full((e_pad - e,), _N, jnp.int32)
    src = jnp.concatenate([edge_index[0], pad])
    dst = jnp.concatenate([edge_index[1], pad])

    zeros1 = jnp.zeros((_NPAD,), jnp.float32)
    ones_k = jnp.ones((_K,), jnp.float32)
    zeros2 = jnp.zeros((_KA, _D), jnp.float32)

    degp = _make_deg_kernel(e_pad)(dst, zeros1, ones_k)
    h = _mm_call(node_feature, W1)
    degt = degp.T
    dinv, hs = _scale_call(h, degt)
    aggp = _make_agg_kernel(e_pad)(hs, src, dst, zeros2)
    prob, rsu = _dense2_call(aggp, hs, dinv, b1, gamma, beta, W2, b2)
    return (prob, rsu)


# speedup vs baseline: 1.2932x; 1.0006x over previous
import jax
import jax.numpy as jnp
from jax import lax
from jax.experimental import pallas as pl
from jax.experimental.pallas import tpu as pltpu
from jax.experimental.pallas import tpu_sc as plsc

_N = 10000
_D = 128
_O = 2
_NC = 2
_NS = 16
_K = 128
_NPAD = 10240
_RPT = _NPAD // _NS


def _sc_mesh():
    return plsc.VectorSubcoreMesh(core_axis_name="c", subcore_axis_name="s",
                                  num_cores=_NC, num_subcores=_NS)


def _make_deg_kernel(e_pad):
    cpt = e_pad // (_NC * _NS * _K)

    def body(dst_hbm, zeros_hbm, ones_hbm, deg_out, d_all, ones_v, deg_sh):
        c = lax.axis_index("c")
        s = lax.axis_index("s")
        r0 = s * _RPT
        base = (c * _NS + s) * cpt * _K
        pltpu.sync_copy(zeros_hbm.at[pl.ds(r0, _RPT)], deg_sh.at[pl.ds(r0, _RPT)])
        pltpu.sync_copy(ones_hbm, ones_v)
        pltpu.sync_copy(dst_hbm.at[pl.ds(base, cpt * _K)], d_all)
        plsc.subcore_barrier()

        def chunk(i, carry):
            pltpu.sync_copy(ones_v, deg_sh.at[d_all.at[pl.ds(i * _K, _K)]],
                            add=True)
            return carry

        lax.fori_loop(0, cpt, chunk, 0)
        plsc.subcore_barrier()
        pltpu.sync_copy(deg_sh.at[pl.ds(r0, _RPT)],
                        deg_out.at[c, pl.ds(r0, _RPT)])

    return pl.kernel(
        body,
        out_type=jax.ShapeDtypeStruct((_NC, _NPAD), jnp.float32),
        mesh=_sc_mesh(),
        scratch_types=[
            pltpu.VMEM((cpt * _K,), jnp.int32),
            pltpu.VMEM((_K,), jnp.float32),
            pltpu.VMEM_SHARED((_NPAD,), jnp.float32),
        ],
    )


def _mm_body(x_ref, w1_ref, h_ref):
    h_ref[0:_N, :] = jnp.dot(x_ref[...], w1_ref[...],
                             preferred_element_type=jnp.float32)
    h_ref[_N:_NPAD, :] = jnp.zeros((_NPAD - _N, _D), jnp.float32)


def _mm_call(x, w1):
    return pl.pallas_call(
        _mm_body,
        out_shape=jax.ShapeDtypeStruct((_NPAD, _D), jnp.float32),
    )(x, w1)


def _scale_body(h_ref, degt_ref, dinv_ref, hs_ref):
    deg = degt_ref[:, 0:1] + degt_ref[:, 1:2] + 1.0
    dinv = lax.rsqrt(deg)
    dinv_ref[...] = dinv
    hs_ref[...] = h_ref[...] * dinv


def _scale_call(h, degt):
    return pl.pallas_call(
        _scale_body,
        out_shape=(
            jax.ShapeDtypeStruct((_NPAD, 1), jnp.float32),
            jax.ShapeDtypeStruct((_NPAD, _D), jnp.float32),
        ),
    )(h, degt)


_KA = 64


def _make_agg_kernel(e_pad):
    cpt = e_pad // (_NC * _NS * _KA)

    def body(hs_hbm, src_hbm, dst_hbm, zeros2_hbm, agg_out,
             s_all, d_all, rows0, rows1, agg_sh, g0, g1):
        c = lax.axis_index("c")
        s = lax.axis_index("s")
        base = (c * _NS + s) * cpt * _KA
        pltpu.sync_copy(zeros2_hbm, rows0)
        for j in range(_RPT // _KA):
            pltpu.sync_copy(rows0, agg_sh.at[pl.ds(s * _RPT + j * _KA, _KA)])
        plsc.subcore_barrier()

        pltpu.sync_copy(src_hbm.at[pl.ds(base, cpt * _KA)], s_all)
        pltpu.sync_copy(dst_hbm.at[pl.ds(base, cpt * _KA)], d_all)

        pltpu.async_copy(hs_hbm.at[s_all.at[pl.ds(0, _KA)]], rows0, g0)

        def pair(j, carry):
            o0 = 2 * j * _KA
            o1 = o0 + _KA
            o2 = jnp.minimum(o1 + _KA, (cpt - 1) * _KA)
            pltpu.make_async_copy(
                hs_hbm.at[s_all.at[pl.ds(o0, _KA)]], rows0, g0).wait()
            pltpu.async_copy(hs_hbm.at[s_all.at[pl.ds(o1, _KA)]], rows1, g1)
            pltpu.sync_copy(rows0, agg_sh.at[d_all.at[pl.ds(o0, _KA)]],
                            add=True)
            pltpu.make_async_copy(
                hs_hbm.at[s_all.at[pl.ds(o1, _KA)]], rows1, g1).wait()
            pltpu.async_copy(hs_hbm.at[s_all.at[pl.ds(o2, _KA)]], rows0, g0)
            pltpu.sync_copy(rows1, agg_sh.at[d_all.at[pl.ds(o1, _KA)]],
                            add=True)
            return carry

        lax.fori_loop(0, cpt // 2, pair, 0)
        pltpu.make_async_copy(
            hs_hbm.at[s_all.at[pl.ds(0, _KA)]], rows0, g0).wait()
        plsc.subcore_barrier()
        r0 = s * _RPT
        pltpu.sync_copy(agg_sh.at[pl.ds(r0, _RPT)],
                        agg_out.at[c, pl.ds(r0, _RPT)])

    def make(cpt):
        return pl.kernel(
            body,
            out_type=jax.ShapeDtypeStruct((_NC, _NPAD, _D), jnp.float32),
            mesh=_sc_mesh(),
            scratch_types=[
                pltpu.VMEM((cpt * _KA,), jnp.int32),
                pltpu.VMEM((cpt * _KA,), jnp.int32),
                pltpu.VMEM((_KA, _D), jnp.float32),
                pltpu.VMEM((_KA, _D), jnp.float32),
                pltpu.VMEM_SHARED((_NPAD, _D), jnp.float32),
                pltpu.SemaphoreType.DMA,
                pltpu.SemaphoreType.DMA,
            ],
        )

    return make(cpt)


def _dense2_body(aggp_ref, hs_ref, dinv_ref, b1_ref, gamma_ref, beta_ref,
                 w2_ref, b2_ref, prob_ref, rsu_ref):
    a = aggp_ref[0, 0:_N, :] + aggp_ref[1, 0:_N, :] + hs_ref[0:_N, :]
    y = a * dinv_ref[0:_N, :] + b1_ref[...]
    mean = jnp.mean(y, axis=0, keepdims=True)
    d = y - mean
    var = jnp.mean(d * d, axis=0, keepdims=True)
    bn = d * lax.rsqrt(var + 1e-5) * gamma_ref[...] + beta_ref[...]
    rsu_ref[...] = bn[0:1, :]
    z = jnp.dot(bn, w2_ref[...], preferred_element_type=jnp.float32)
    z = jnp.maximum(z + b2_ref[...], 0.0)
    m = jnp.max(z, axis=1, keepdims=True)
    e = jnp.exp(z - m)
    prob_ref[...] = e / jnp.sum(e, axis=1, keepdims=True)


def _dense2_call(aggp, hs, dinv, b1, gamma, beta, w2, b2):
    return pl.pallas_call(
        _dense2_body,
        out_shape=(
            jax.ShapeDtypeStruct((_N, _O), jnp.float32),
            jax.ShapeDtypeStruct((1, _D), jnp.float32),
        ),
    )(aggp, hs, dinv, b1, gamma, beta, w2, b2)


def kernel(node_feature, edge_index, W1, b1, gamma, beta, W2, b2):
    e = edge_index.shape[1]
    chunk = _NC * _NS * _K
    e_pad = ((e + chunk - 1) // chunk) * chunk
    pad_dst = _N + (jnp.arange(e_pad - e, dtype=jnp.int32) % (_NPAD - _N))
    pad_src = jnp.full((e_pad - e,), _N, jnp.int32)
    src = jnp.concatenate([edge_index[0], pad_src])
    dst = jnp.concatenate([edge_index[1], pad_dst])

    zeros1 = jnp.zeros((_NPAD,), jnp.float32)
    ones_k = jnp.ones((_K,), jnp.float32)
    zeros2 = jnp.zeros((_KA, _D), jnp.float32)

    degp = _make_deg_kernel(e_pad)(dst, zeros1, ones_k)
    h = _mm_call(node_feature, W1)
    degt = degp.T
    dinv, hs = _scale_call(h, degt)
    aggp = _make_agg_kernel(e_pad)(hs, src, dst, zeros2)
    prob, rsu = _dense2_call(aggp, hs, dinv, b1, gamma, beta, W2, b2)
    return (prob, rsu)
